# baseline (speedup 1.0000x reference)
import jax
import jax.numpy as jnp
from jax.experimental import pallas as pl
from jax.experimental.pallas import tpu as pltpu

N = 32768
C = 64
M = 1024
BLK = 512
CH = 16


def _vq_kernel(qn_ref, k_ref, kt_ref, z_ref, idx_ref, oh_ref, cnt_ref, perp_ref,
               sim_ref):
    i = pl.program_id(0)
    nblocks = pl.num_programs(0)

    sim_ref[...] = jnp.dot(qn_ref[...], kt_ref[...],
                           preferred_element_type=jnp.float32)
    l2k = jnp.sum(kt_ref[...] * kt_ref[...], axis=0, keepdims=True)

    def body(s, _):
        rs = CH * s
        qc = qn_ref[pl.ds(rs, CH), :]
        l2q = 0.25 * jnp.sum(qc * qc, axis=1, keepdims=True)
        d = (l2q + l2k) + sim_ref[pl.ds(rs, CH), :]
        mval = jnp.min(d, axis=1, keepdims=True)
        lane = jax.lax.broadcasted_iota(jnp.int32, d.shape, 1)
        idx = jnp.min(jnp.where(d == mval, lane, M), axis=1, keepdims=True)
        idx_ref[pl.ds(rs, CH), :] = idx
        oh_ref[pl.ds(rs, CH), :] = (lane == idx).astype(jnp.float32)
        return _

    jax.lax.fori_loop(0, BLK // CH, body, None, unroll=False)

    onehot = oh_ref[...]
    z_ref[...] = jnp.dot(onehot, k_ref[...], preferred_element_type=jnp.float32)
    ones = jnp.full((8, BLK // 8), 1.0, jnp.float32).reshape(1, BLK)
    part = jnp.dot(ones, onehot, preferred_element_type=jnp.float32)

    @pl.when(i == 0)
    def _init():
        cnt_ref[...] = part

    @pl.when(i != 0)
    def _acc():
        cnt_ref[...] += part

    @pl.when(i == nblocks - 1)
    def _finish():
        p = cnt_ref[...] * (1.0 / N)
        s = jnp.sum(p * jnp.log(p + 1e-10), axis=1, keepdims=True)
        perp_ref[...] = jnp.exp(-s)


@jax.jit
def kernel(q, k):
    qn = q * (-2.0)
    kt = k.T
    grid = (N // BLK,)
    z, idx, onehot, _cnt, perp = pl.pallas_call(
        _vq_kernel,
        grid=grid,
        in_specs=[
            pl.BlockSpec((BLK, C), lambda i: (i, 0)),
            pl.BlockSpec((M, C), lambda i: (0, 0)),
            pl.BlockSpec((C, M), lambda i: (0, 0)),
        ],
        out_specs=[
            pl.BlockSpec((BLK, C), lambda i: (i, 0)),
            pl.BlockSpec((BLK, 1), lambda i: (i, 0)),
            pl.BlockSpec((BLK, M), lambda i: (i, 0)),
            pl.BlockSpec((1, M), lambda i: (0, 0)),
            pl.BlockSpec((1, 1), lambda i: (0, 0)),
        ],
        out_shape=[
            jax.ShapeDtypeStruct((N, C), jnp.float32),
            jax.ShapeDtypeStruct((N, 1), jnp.int32),
            jax.ShapeDtypeStruct((N, M), jnp.float32),
            jax.ShapeDtypeStruct((1, M), jnp.float32),
            jax.ShapeDtypeStruct((1, 1), jnp.float32),
        ],
        scratch_shapes=[pltpu.VMEM((BLK, M), jnp.float32)],
        compiler_params=pltpu.CompilerParams(
            dimension_semantics=("arbitrary",),
        ),
    )(qn, k, kt)
    return (z, idx.reshape(N), onehot, perp[0, 0])


# speedup vs baseline: 4.7416x; 4.7416x over previous
import jax
import jax.numpy as jnp
from jax.experimental import pallas as pl
from jax.experimental.pallas import tpu as pltpu

N = 32768
C = 64
M = 1024
BLK = 512
CH = 16


def _vq_kernel(qn_ref, k_ref, kt_ref, z_ref, idx_ref, oh_ref, cnt_ref, perp_ref,
               sim_ref):
    i = pl.program_id(0)
    nblocks = pl.num_programs(0)

    sim_ref[...] = jnp.dot(qn_ref[...], kt_ref[...],
                           preferred_element_type=jnp.float32)
    l2k = jnp.sum(kt_ref[...] * kt_ref[...], axis=0, keepdims=True)

    for s in range(BLK // CH):
        rs = CH * s
        qc = qn_ref[pl.ds(rs, CH), :]
        l2q = 0.25 * jnp.sum(qc * qc, axis=1, keepdims=True)
        d = (l2q + l2k) + sim_ref[pl.ds(rs, CH), :]
        mval = jnp.min(d, axis=1, keepdims=True)
        lane = jax.lax.broadcasted_iota(jnp.int32, d.shape, 1)
        idx = jnp.min(jnp.where(d == mval, lane, M), axis=1, keepdims=True)
        idx_ref[pl.ds(rs, CH), :] = idx
        oh_ref[pl.ds(rs, CH), :] = (lane == idx).astype(jnp.float32)

    onehot = oh_ref[...]
    z_ref[...] = jnp.dot(onehot, k_ref[...], preferred_element_type=jnp.float32)
    ones = jnp.full((8, BLK // 8), 1.0, jnp.float32).reshape(1, BLK)
    part = jnp.dot(ones, onehot, preferred_element_type=jnp.float32)

    @pl.when(i == 0)
    def _init():
        cnt_ref[...] = part

    @pl.when(i != 0)
    def _acc():
        cnt_ref[...] += part

    @pl.when(i == nblocks - 1)
    def _finish():
        p = cnt_ref[...] * (1.0 / N)
        s = jnp.sum(p * jnp.log(p + 1e-10), axis=1, keepdims=True)
        perp_ref[...] = jnp.exp(-s)


@jax.jit
def kernel(q, k):
    qn = q * (-2.0)
    kt = k.T
    grid = (N // BLK,)
    z, idx, onehot, _cnt, perp = pl.pallas_call(
        _vq_kernel,
        grid=grid,
        in_specs=[
            pl.BlockSpec((BLK, C), lambda i: (i, 0)),
            pl.BlockSpec((M, C), lambda i: (0, 0)),
            pl.BlockSpec((C, M), lambda i: (0, 0)),
        ],
        out_specs=[
            pl.BlockSpec((BLK, C), lambda i: (i, 0)),
            pl.BlockSpec((BLK, 1), lambda i: (i, 0)),
            pl.BlockSpec((BLK, M), lambda i: (i, 0)),
            pl.BlockSpec((1, M), lambda i: (0, 0)),
            pl.BlockSpec((1, 1), lambda i: (0, 0)),
        ],
        out_shape=[
            jax.ShapeDtypeStruct((N, C), jnp.float32),
            jax.ShapeDtypeStruct((N, 1), jnp.int32),
            jax.ShapeDtypeStruct((N, M), jnp.float32),
            jax.ShapeDtypeStruct((1, M), jnp.float32),
            jax.ShapeDtypeStruct((1, 1), jnp.float32),
        ],
        scratch_shapes=[pltpu.VMEM((BLK, M), jnp.float32)],
        compiler_params=pltpu.CompilerParams(
            dimension_semantics=("arbitrary",),
        ),
    )(qn, k, kt)
    return (z, idx.reshape(N), onehot, perp[0, 0])


# speedup vs baseline: 4.7495x; 1.0017x over previous
import jax
import jax.numpy as jnp
from jax.experimental import pallas as pl
from jax.experimental.pallas import tpu as pltpu

N = 32768
C = 64
M = 1024
BLK = 512


def _vq_kernel(qn_ref, k_ref, kt_ref, ones_ref, z_ref, idx_ref, oh_ref, cnt_ref, perp_ref):
    i = pl.program_id(0)
    nblocks = pl.num_programs(0)

    qb = qn_ref[...]
    kt = kt_ref[...]

    l2q = 0.25 * jnp.sum(qb * qb, axis=1, keepdims=True)
    l2k = jnp.sum(kt * kt, axis=0, keepdims=True)
    simn = jnp.dot(qb, kt, preferred_element_type=jnp.float32)
    t = l2q + l2k
    dist = t + simn

    mval = jnp.min(dist, axis=1, keepdims=True)
    lane = jax.lax.broadcasted_iota(jnp.int32, dist.shape, 1)
    idx = jnp.min(jnp.where(dist == mval, lane, M), axis=1, keepdims=True)

    onehot = (lane == idx).astype(jnp.float32)
    oh_ref[...] = onehot
    idx_ref[...] = idx
    z_ref[...] = jnp.dot(onehot, k_ref[...], preferred_element_type=jnp.float32)

    part = jnp.dot(ones_ref[...], onehot, preferred_element_type=jnp.float32)

    @pl.when(i == 0)
    def _init():
        cnt_ref[...] = part

    @pl.when(i != 0)
    def _acc():
        cnt_ref[...] += part

    @pl.when(i == nblocks - 1)
    def _finish():
        p = cnt_ref[...] * (1.0 / N)
        s = jnp.sum(p * jnp.log(p + 1e-10), axis=1, keepdims=True)
        perp_ref[...] = jnp.exp(-s)


@jax.jit
def kernel(q, k):
    qn = q * (-2.0)
    kt = k.T
    ones = jnp.ones((1, BLK), jnp.float32)
    grid = (N // BLK,)
    z, idx, onehot, _cnt, perp = pl.pallas_call(
        _vq_kernel,
        grid=grid,
        in_specs=[
            pl.BlockSpec((BLK, C), lambda i: (i, 0)),
            pl.BlockSpec((M, C), lambda i: (0, 0)),
            pl.BlockSpec((C, M), lambda i: (0, 0)),
            pl.BlockSpec((1, BLK), lambda i: (0, 0)),
        ],
        out_specs=[
            pl.BlockSpec((BLK, C), lambda i: (i, 0)),
            pl.BlockSpec((BLK, 1), lambda i: (i, 0)),
            pl.BlockSpec((BLK, M), lambda i: (i, 0)),
            pl.BlockSpec((1, M), lambda i: (0, 0)),
            pl.BlockSpec((1, 1), lambda i: (0, 0)),
        ],
        out_shape=[
            jax.ShapeDtypeStruct((N, C), jnp.float32),
            jax.ShapeDtypeStruct((N, 1), jnp.int32),
            jax.ShapeDtypeStruct((N, M), jnp.float32),
            jax.ShapeDtypeStruct((1, M), jnp.float32),
            jax.ShapeDtypeStruct((1, 1), jnp.float32),
        ],
        compiler_params=pltpu.CompilerParams(
            dimension_semantics=("arbitrary",),
        ),
    )(qn, k, kt, ones)
    return (z, idx.reshape(N), onehot, perp[0, 0])


# speedup vs baseline: 4.8871x; 1.0290x over previous
import jax
import jax.numpy as jnp
from jax.experimental import pallas as pl
from jax.experimental.pallas import tpu as pltpu

N = 32768
C = 64
M = 1024
BLK = 512


def _vq_kernel(qn_ref, k_ref, kt_ref, ones_ref, z_ref, idx_ref, oh_ref, cnt_ref, perp_ref):
    i = pl.program_id(0)
    nblocks = pl.num_programs(0)

    qb = qn_ref[...]
    kt = kt_ref[...]

    l2q = 0.25 * jnp.sum(qb * qb, axis=1, keepdims=True)
    l2k = jnp.sum(kt * kt, axis=0, keepdims=True)
    simn = jnp.dot(qb, kt, preferred_element_type=jnp.float32)
    t = l2q + l2k
    dist = t + simn

    mval = jnp.min(dist, axis=1, keepdims=True)
    lane = jax.lax.broadcasted_iota(jnp.int32, dist.shape, 1)
    idx = jnp.min(jnp.where(dist == mval, lane, M), axis=1, keepdims=True)

    onehot = (lane == idx).astype(jnp.float32)
    oh_ref[...] = onehot
    idx_ref[...] = idx
    z_ref[...] = jnp.dot(onehot, k_ref[...], preferred_element_type=jnp.float32)

    part = jnp.sum(onehot, axis=0, keepdims=True)

    @pl.when(i == 0)
    def _init():
        cnt_ref[...] = part

    @pl.when(i != 0)
    def _acc():
        cnt_ref[...] += part

    @pl.when(i == nblocks - 1)
    def _finish():
        p = cnt_ref[...] * (1.0 / N)
        s = jnp.sum(p * jnp.log(p + 1e-10), axis=1, keepdims=True)
        perp_ref[...] = jnp.exp(-s)


@jax.jit
def kernel(q, k):
    qn = q * (-2.0)
    kt = k.T
    ones = jnp.ones((1, BLK), jnp.float32)
    grid = (N // BLK,)
    z, idx, onehot, _cnt, perp = pl.pallas_call(
        _vq_kernel,
        grid=grid,
        in_specs=[
            pl.BlockSpec((BLK, C), lambda i: (i, 0)),
            pl.BlockSpec((M, C), lambda i: (0, 0)),
            pl.BlockSpec((C, M), lambda i: (0, 0)),
            pl.BlockSpec((1, BLK), lambda i: (0, 0)),
        ],
        out_specs=[
            pl.BlockSpec((BLK, C), lambda i: (i, 0)),
            pl.BlockSpec((BLK, 1), lambda i: (i, 0)),
            pl.BlockSpec((BLK, M), lambda i: (i, 0)),
            pl.BlockSpec((1, M), lambda i: (0, 0)),
            pl.BlockSpec((1, 1), lambda i: (0, 0)),
        ],
        out_shape=[
            jax.ShapeDtypeStruct((N, C), jnp.float32),
            jax.ShapeDtypeStruct((N, 1), jnp.int32),
            jax.ShapeDtypeStruct((N, M), jnp.float32),
            jax.ShapeDtypeStruct((1, M), jnp.float32),
            jax.ShapeDtypeStruct((1, 1), jnp.float32),
        ],
        compiler_params=pltpu.CompilerParams(
            dimension_semantics=("arbitrary",),
        ),
    )(qn, k, kt, ones)
    return (z, idx.reshape(N), onehot, perp[0, 0])


# speedup vs baseline: 5.7044x; 1.1672x over previous
import jax
import jax.numpy as jnp
from jax.experimental import pallas as pl
from jax.experimental.pallas import tpu as pltpu

N = 32768
C = 64
M = 1024
BLK = 512


def _vq_kernel(q_ref, k_ref, ktn_ref, z_ref, idx_ref, oh_ref, cnt_ref, perp_ref):
    i = pl.program_id(0)
    nblocks = pl.num_programs(0)

    qb = q_ref[...]
    ktn = ktn_ref[...]

    l2q = jnp.sum(qb * qb, axis=1, keepdims=True)
    l2k = 0.25 * jnp.sum(ktn * ktn, axis=0, keepdims=True)
    simn = jnp.dot(qb, ktn, preferred_element_type=jnp.float32)
    dist = (l2q + l2k) + simn

    mval = jnp.min(dist, axis=1, keepdims=True)
    lane = jax.lax.broadcasted_iota(jnp.int32, dist.shape, 1)
    idx = jnp.min(jnp.where(dist == mval, lane, M), axis=1, keepdims=True)

    onehot = (lane == idx).astype(jnp.float32)
    oh_ref[...] = onehot
    idx_ref[...] = idx
    z_ref[...] = jnp.dot(onehot, k_ref[...], preferred_element_type=jnp.float32)

    part = jnp.sum(onehot, axis=0, keepdims=True)

    @pl.when(i == 0)
    def _init():
        cnt_ref[...] = part

    @pl.when(i != 0)
    def _acc():
        cnt_ref[...] += part

    @pl.when(i == nblocks - 1)
    def _finish():
        p = cnt_ref[...] * (1.0 / N)
        s = jnp.sum(p * jnp.log(p + 1e-10), axis=1, keepdims=True)
        perp_ref[...] = jnp.exp(-s)


@jax.jit
def kernel(q, k):
    ktn = k.T * (-2.0)
    grid = (N // BLK,)
    z, idx, onehot, _cnt, perp = pl.pallas_call(
        _vq_kernel,
        grid=grid,
        in_specs=[
            pl.BlockSpec((BLK, C), lambda i: (i, 0)),
            pl.BlockSpec((M, C), lambda i: (0, 0)),
            pl.BlockSpec((C, M), lambda i: (0, 0)),
        ],
        out_specs=[
            pl.BlockSpec((BLK, C), lambda i: (i, 0)),
            pl.BlockSpec((BLK, 1), lambda i: (i, 0)),
            pl.BlockSpec((BLK, M), lambda i: (i, 0)),
            pl.BlockSpec((1, M), lambda i: (0, 0)),
            pl.BlockSpec((1, 1), lambda i: (0, 0)),
        ],
        out_shape=[
            jax.ShapeDtypeStruct((N, C), jnp.float32),
            jax.ShapeDtypeStruct((N, 1), jnp.int32),
            jax.ShapeDtypeStruct((N, M), jnp.float32),
            jax.ShapeDtypeStruct((1, M), jnp.float32),
            jax.ShapeDtypeStruct((1, 1), jnp.float32),
        ],
        compiler_params=pltpu.CompilerParams(
            dimension_semantics=("arbitrary",),
        ),
    )(q, k, ktn)
    return (z, idx.reshape(N), onehot, perp[0, 0])


# speedup vs baseline: 6.2234x; 1.0910x over previous
import jax
import jax.numpy as jnp
from jax.experimental import pallas as pl
from jax.experimental.pallas import tpu as pltpu

N = 32768
C = 64
M = 1024
BLK = 1024


def _vq_kernel(q_ref, k_ref, ktn_ref, z_ref, idx_ref, oh_ref, cnt_ref, perp_ref):
    i = pl.program_id(0)
    nblocks = pl.num_programs(0)

    qb = q_ref[...]
    ktn = ktn_ref[...]

    l2q = jnp.sum(qb * qb, axis=1, keepdims=True)
    l2k = 0.25 * jnp.sum(ktn * ktn, axis=0, keepdims=True)
    simn = jnp.dot(qb, ktn, preferred_element_type=jnp.float32)
    dist = (l2q + l2k) + simn

    mval = jnp.min(dist, axis=1, keepdims=True)
    lane = jax.lax.broadcasted_iota(jnp.int32, dist.shape, 1)
    idx = jnp.min(jnp.where(dist == mval, lane, M), axis=1, keepdims=True)

    onehot = (lane == idx).astype(jnp.float32)
    oh_ref[...] = onehot
    idx_ref[...] = idx
    z_ref[...] = jnp.dot(onehot, k_ref[...], preferred_element_type=jnp.float32)

    part = jnp.sum(onehot, axis=0, keepdims=True)

    @pl.when(i == 0)
    def _init():
        cnt_ref[...] = part

    @pl.when(i != 0)
    def _acc():
        cnt_ref[...] += part

    @pl.when(i == nblocks - 1)
    def _finish():
        p = cnt_ref[...] * (1.0 / N)
        s = jnp.sum(p * jnp.log(p + 1e-10), axis=1, keepdims=True)
        perp_ref[...] = jnp.exp(-s)


@jax.jit
def kernel(q, k):
    ktn = k.T * (-2.0)
    grid = (N // BLK,)
    z, idx, onehot, _cnt, perp = pl.pallas_call(
        _vq_kernel,
        grid=grid,
        in_specs=[
            pl.BlockSpec((BLK, C), lambda i: (i, 0)),
            pl.BlockSpec((M, C), lambda i: (0, 0)),
            pl.BlockSpec((C, M), lambda i: (0, 0)),
        ],
        out_specs=[
            pl.BlockSpec((BLK, C), lambda i: (i, 0)),
            pl.BlockSpec((BLK, 1), lambda i: (i, 0)),
            pl.BlockSpec((BLK, M), lambda i: (i, 0)),
            pl.BlockSpec((1, M), lambda i: (0, 0)),
            pl.BlockSpec((1, 1), lambda i: (0, 0)),
        ],
        out_shape=[
            jax.ShapeDtypeStruct((N, C), jnp.float32),
            jax.ShapeDtypeStruct((N, 1), jnp.int32),
            jax.ShapeDtypeStruct((N, M), jnp.float32),
            jax.ShapeDtypeStruct((1, M), jnp.float32),
            jax.ShapeDtypeStruct((1, 1), jnp.float32),
        ],
        compiler_params=pltpu.CompilerParams(
            dimension_semantics=("arbitrary",),
        ),
    )(q, k, ktn)
    return (z, idx.reshape(N), onehot, perp[0, 0])
